# baseline (speedup 1.0000x reference)
import functools

import numpy as np
import jax
import jax.numpy as jnp
from jax import lax
from jax.experimental import pallas as pl
from jax.experimental.pallas import tpu as pltpu
from jax.experimental.pallas import tpu_sc as plsc

BB, CC, HH, WW = 2, 128, 96, 96
N = HH * WW
K = 1024
HI = 384
HID = 128
NSUB = 16
RPT = HH // NSUB


def _interp_matrix():
    ys = np.linspace(0.0, HI - 1.0, HH)
    y0 = np.clip(np.floor(ys).astype(np.int32), 0, HI - 1)
    y1 = np.clip(y0 + 1, 0, HI - 1)
    wy = (ys - y0).astype(np.float32)
    ry = np.zeros((HH, HI), np.float32)
    ar = np.arange(HH)
    np.add.at(ry, (ar, y0), 1.0 - wy)
    np.add.at(ry, (ar, y1), wy)
    return ry


def _bilinear_body(img_ref, ry_ref, rxt_ref, out_ref):
    ch = img_ref[0, 0]
    e = jnp.dot(ry_ref[...], ch, preferred_element_type=jnp.float32)
    out_ref[0, 0] = jnp.dot(e, rxt_ref[...],
                            preferred_element_type=jnp.float32)


def _rsqrt_vec(s):
    i = plsc.bitcast(s, jnp.int32)
    y = plsc.bitcast(jnp.int32(0x5F3759DF) - (i >> 1), jnp.float32)
    for _ in range(3):
        y = y * (1.5 - 0.5 * s * y * y)
    return y


def _fill1d(ref, n, val):
    v = jnp.full((16,), val, ref.dtype)

    def bd(i, c):
        for q in range(4):
            ref[pl.ds((i * 4 + q) * 16, 16)] = v
        return c

    lax.fori_loop(0, n // 64, bd, 0)


def _fill2d(ref, rows):
    v = jnp.zeros((16,), ref.dtype)

    def bd(i, c):
        for q in range(8):
            ref[i, pl.ds(q * 16, 16)] = v
        return c

    lax.fori_loop(0, rows, bd, 0)


def _scale_rows(ref, sref, srow, nrows):
    def bd(g, c):
        sv = sref[srow, pl.ds(g * 16, 16)]
        for jj in range(16):
            s = sv[jj]
            i = g * 16 + jj
            for q in range(8):
                sl = pl.ds(q * 16, 16)
                ref[i, sl] = ref[i, sl] * s
        return c

    lax.fori_loop(0, nrows // 16, bd, 0)


def _sc_ab_body(colors, seg4, xT,
                node_pre, validf, simout, adjcnt,
                sh_sr, sh_sg, sh_sb, sh_cnt, sh_ws, sh_node, sh_adj,
                v_cid, v_col, v_ones, v_mr, v_mg, v_mb, v_cnt, v_valid,
                v_sim, v_wsq, v_feat0, v_feat1, v_aidx, v_aval, v_z1, v_b64,
                v_segrow, sem_z, sem_a, sem_in, sem_sc, sem_f0, sem_f1,
                sem_d):
    t = lax.axis_index("s")
    b = lax.axis_index("c")
    r0 = t * RPT
    iota16 = lax.iota(jnp.int32, 16)
    feat_bufs = (v_feat0, v_feat1)

    with jax.named_scope("ab_zero"):
        _fill1d(v_z1, 8192, 0.0)
        _fill2d(v_b64, 64)
        one = jnp.full((16,), 1.0, jnp.float32)
        for q in range(6):
            v_ones[pl.ds(q * 16, 16)] = one

        zd = []
        for j in range(8):
            zd.append(pltpu.async_copy(
                v_z1, sh_adj.at[pl.ds(t * 65536 + j * 8192, 8192)], sem_z))
        zd.append(pltpu.async_copy(v_b64, sh_node.at[pl.ds(t * 64, 64)],
                                   sem_z))

        @pl.when(t == 0)
        def _():
            for sh in (sh_sr, sh_sg, sh_sb, sh_cnt, sh_ws):
                pltpu.sync_copy(v_z1.at[pl.ds(0, K)], sh)

        for d in zd:
            d.wait()
        plsc.subcore_barrier()

    with jax.named_scope("ab_stage_a"):
        sd = []
        for r in range(RPT):
            sd.append(pltpu.async_copy(seg4.at[b, r0 + r, 0],
                                       v_segrow.at[r], sem_in))
        for c in range(3):
            sd.append(pltpu.async_copy(colors.at[b, c, t], v_col.at[c],
                                       sem_in))

        @pl.when(t < NSUB - 1)
        def _():
            pltpu.sync_copy(seg4.at[b, (t + 1) * RPT, 0], v_segrow.at[RPT])

        @pl.when(t == NSUB - 1)
        def _():
            z = jnp.zeros((16,), jnp.int32)
            for u in range(6):
                v_cid[RPT, pl.ds(u * 16, 16)] = z

        for d in sd:
            d.wait()
        for r in range(RPT):
            for u in range(6):
                cv = plsc.load_gather(v_segrow.at[r], [iota16 * 4 + u * 64])
                v_cid[r, pl.ds(u * 16, 16)] = cv

        @pl.when(t < NSUB - 1)
        def _():
            for u in range(6):
                cv = plsc.load_gather(v_segrow.at[RPT], [iota16 * 4 + u * 64])
                v_cid[RPT, pl.ds(u * 16, 16)] = cv
        ad = []
        for r in range(RPT):
            idx = v_cid.at[r]
            ad.append(pltpu.async_copy(v_col.at[0, r], sh_sr.at[idx], sem_a,
                                       add=True))
            ad.append(pltpu.async_copy(v_col.at[1, r], sh_sg.at[idx], sem_a,
                                       add=True))
            ad.append(pltpu.async_copy(v_col.at[2, r], sh_sb.at[idx], sem_a,
                                       add=True))
            ad.append(pltpu.async_copy(v_ones, sh_cnt.at[idx], sem_a,
                                       add=True))
        for d in ad:
            d.wait()
        plsc.subcore_barrier()

    with jax.named_scope("ab_means"):
        pltpu.sync_copy(sh_sr, v_mr)
        pltpu.sync_copy(sh_sg, v_mg)
        pltpu.sync_copy(sh_sb, v_mb)
        pltpu.sync_copy(sh_cnt, v_cnt)

        def mean_bd(i, c):
            sl = pl.ds(i * 16, 16)
            cnt = v_cnt[sl]
            rc = 1.0 / (cnt + 1e-8)
            v_mr[sl] = v_mr[sl] * rc
            v_mg[sl] = v_mg[sl] * rc
            v_mb[sl] = v_mb[sl] * rc
            v_valid[sl] = jnp.where(cnt > 0.0, 1.0, 0.0)
            return c

        lax.fori_loop(0, K // 16, mean_bd, 0)

        @pl.when(t == 0)
        def _():
            pltpu.sync_copy(v_valid, validf.at[b, 0])

    with jax.named_scope("ab_rows"):
        feat_in = [None, None]
        feat_sc = [None, None]
        feat_in[0] = pltpu.async_copy(xT.at[pl.ds(b * N + r0 * 96, 96)],
                                      feat_bufs[0], sem_in)
        row_descs = [None] * RPT
        for r in range(RPT):
            gr = r0 + r
            p = r % 2
            rowfull = jnp.full((16,), r, jnp.int32)
            feat_in[p].wait()
            if r + 1 < RPT:
                if feat_sc[1 - p] is not None:
                    feat_sc[1 - p].wait()
                feat_in[1 - p] = pltpu.async_copy(
                    xT.at[pl.ds(b * N + (gr + 1) * 96, 96)],
                    feat_bufs[1 - p], sem_in)
            for u in range(6):
                sl = pl.ds(u * 16, 16)
                cidv = v_cid[r, sl]
                cr = v_col[0, r, sl]
                cg = v_col[1, r, sl]
                cb = v_col[2, r, sl]
                mr = plsc.load_gather(v_mr, [cidv])
                mg = plsc.load_gather(v_mg, [cidv])
                mb = plsc.load_gather(v_mb, [cidv])
                dot = cr * mr + cg * mg + cb * mb
                n1 = jnp.maximum(cr * cr + cg * cg + cb * cb, 1e-16)
                n2 = jnp.maximum(mr * mr + mg * mg + mb * mb, 1e-16)
                sim = jnp.clip(dot * _rsqrt_vec(n1) * _rsqrt_vec(n2), 0.0, 1.0)
                v_sim[r, sl] = sim
                v_wsq[r, sl] = sim * sim
                j = iota16 + (u * 16)
                jn = jnp.minimum(j + 1, 95)
                bh = plsc.load_gather(v_cid, [rowfull, jn])
                okh = jnp.logical_and(cidv != bh, j < 95)
                v_aval[2 * r, sl] = jnp.where(okh, 1.0, 0.0)
                v_aidx[2 * r, 0, sl] = cidv * K + bh
                v_aidx[2 * r, 1, sl] = bh * K + cidv
                bv = v_cid[r + 1, sl]
                okv = jnp.logical_and(cidv != bv, gr < HH - 1)
                v_aval[2 * r + 1, sl] = jnp.where(okv, 1.0, 0.0)
                v_aidx[2 * r + 1, 0, sl] = cidv * K + bv
                v_aidx[2 * r + 1, 1, sl] = bv * K + cidv
            rd = []
            for hv in range(2):
                for d2 in range(2):
                    rd.append(pltpu.async_copy(
                        v_aval.at[2 * r + hv],
                        sh_adj.at[v_aidx.at[2 * r + hv, d2]], sem_sc,
                        add=True))
            rd.append(pltpu.async_copy(v_wsq.at[r], sh_ws.at[v_cid.at[r]],
                                       sem_sc, add=True))
            row_descs[r] = rd
            _scale_rows(feat_bufs[p], v_sim, r, 96)
            feat_sc[p] = pltpu.async_copy(feat_bufs[p],
                                          sh_node.at[v_cid.at[r]],
                                          (sem_f0, sem_f1)[p], add=True)
            if r >= 1:
                for d in row_descs[r - 1]:
                    d.wait()
        for d in row_descs[RPT - 1]:
            d.wait()
        for p in range(2):
            if feat_sc[p] is not None:
                feat_sc[p].wait()
        pltpu.sync_copy(v_sim, simout.at[b, t])
        plsc.subcore_barrier()

    with jax.named_scope("ab_dump"):
        adj_d = pltpu.async_copy(sh_adj.at[pl.ds(t * 65536, 65536)],
                                 adjcnt.at[b, pl.ds(t * 65536, 65536)], sem_d)
        pltpu.sync_copy(sh_node.at[pl.ds(t * 64, 64)], v_b64)
        pltpu.sync_copy(sh_ws.at[pl.ds(t * 64, 64)], v_wsq.at[0, pl.ds(0, 64)])

        def div_bd(g, c):
            wv = v_wsq[0, pl.ds(g * 16, 16)]
            rcv = 1.0 / (wv + 1e-8)
            for jj in range(16):
                rc = rcv[jj]
                i = g * 16 + jj
                for q in range(8):
                    sl = pl.ds(q * 16, 16)
                    v_b64[i, sl] = v_b64[i, sl] * rc
            return c

        lax.fori_loop(0, 4, div_bd, 0)
        pltpu.sync_copy(v_b64, node_pre.at[b, pl.ds(t * 64, 64)])
        adj_d.wait()


def _ln_rows(xv, g, bvec):
    m = jnp.mean(xv, axis=-1, keepdims=True)
    d = xv - m
    v = jnp.mean(d * d, axis=-1, keepdims=True)
    return d * lax.rsqrt(v + 1e-5) * g + bvec


def _dense_body(np_ref, va_ref, adj_ref, pw, pb, gw, gb, aw, ab, aow, aob,
                f1w, f1b, f2w, f2b, l1g, l1b, l2g, l2b, out_ref):
    npre = np_ref[0]
    adjc = adj_ref[0]
    validr = va_ref[0]

    def matr(xv, wref):
        return lax.dot_general(xv, wref[...], (((1,), (1,)), ((), ())),
                               preferred_element_type=jnp.float32)

    node = matr(npre, pw) + pb[...]
    adj = (adjc > 0.0).astype(jnp.float32)
    deg = jnp.sum(adj, axis=1, keepdims=True) + 1.0
    dinv = lax.rsqrt(jnp.maximum(deg, 1e-12))
    xw = matr(node, gw)
    y = dinv * xw
    ay = lax.dot_general(adj, y, (((1,), (0,)), ((), ())),
                         preferred_element_type=jnp.float32)
    node = jax.nn.relu(dinv * (ay + y) + gb[...])

    qkv = matr(node, aw) + ab[...]
    heads = []
    for hh in range(4):
        q = qkv[:, 32 * hh:32 * hh + 32]
        kk = qkv[:, 128 + 32 * hh:128 + 32 * hh + 32]
        vv = qkv[:, 256 + 32 * hh:256 + 32 * hh + 32]
        lg = lax.dot_general(q, kk, (((1,), (1,)), ((), ())),
                             preferred_element_type=jnp.float32)
        lg = lg * (1.0 / np.sqrt(32.0))
        lg = jnp.where(validr > 0.0, lg, -1e30)
        mx = jnp.max(lg, axis=-1, keepdims=True)
        e = jnp.exp(lg - mx)
        s = jnp.sum(e, axis=-1, keepdims=True)
        o_h = lax.dot_general(e, vv, (((1,), (0,)), ((), ())),
                              preferred_element_type=jnp.float32) / s
        heads.append(o_h)
    o = jnp.concatenate(heads, axis=1)
    att = matr(o, aow) + aob[...]
    h1 = _ln_rows(node + att, l1g[...], l1b[...])
    ff = matr(jax.nn.relu(matr(h1, f1w) + f1b[...]), f2w) + f2b[...]
    out_ref[0] = _ln_rows(h1 + ff, l2g[...], l2b[...])


RPO = 3


def _sc_out_body(bb, h2d, sim32, seg4, outT,
                 v_idx, v_sim, v_rows0, v_rows1, v_segrow, sem_g, sem_o0,
                 sem_o1):
    t = lax.axis_index("s")
    c = lax.axis_index("c")
    wid = t * 2 + c
    r0 = wid * RPO
    iota16 = lax.iota(jnp.int32, 16)
    bufs = (v_rows0, v_rows1)
    sd = [pltpu.async_copy(sim32.at[bb, wid], v_sim, sem_g)]
    for r in range(RPO):
        sd.append(pltpu.async_copy(seg4.at[bb, r0 + r, 0], v_segrow.at[r],
                                   sem_g))
    for d in sd:
        d.wait()
    for r in range(RPO):
        for u in range(6):
            cv = plsc.load_gather(v_segrow.at[r], [iota16 * 4 + u * 64])
            v_idx[r, pl.ds(u * 16, 16)] = cv
    gin = [None, None]
    gout = [None, None]
    gin[0] = pltpu.async_copy(h2d.at[v_idx.at[0]], bufs[0], sem_g)
    for r in range(RPO):
        p = r % 2
        gin[p].wait()
        if r + 1 < RPO:
            if gout[1 - p] is not None:
                gout[1 - p].wait()
            gin[1 - p] = pltpu.async_copy(h2d.at[v_idx.at[r + 1]],
                                          bufs[1 - p], sem_g)
        _scale_rows(bufs[p], v_sim, r, 96)
        gout[p] = pltpu.async_copy(
            bufs[p], outT.at[pl.ds((r0 + r) * 96, 96)],
            (sem_o0, sem_o1)[p])
    for p in range(2):
        if gout[p] is not None:
            gout[p].wait()


def kernel(x, img, segments, proj_in_w, proj_in_b, gcn_w, gcn_b, attn_in_w,
           attn_in_b, attn_out_w, attn_out_b, ff1_w, ff1_b, ff2_w, ff2_b,
           ln1_g, ln1_b, ln2_g, ln2_b):
    f32 = jnp.float32
    ry = jnp.asarray(_interp_matrix())
    rxt = jnp.asarray(_interp_matrix().T)

    colors = pl.pallas_call(
        _bilinear_body,
        grid=(BB, 3),
        in_specs=[
            pl.BlockSpec((1, 1, HI, HI), lambda b, c: (b, c, 0, 0)),
            pl.BlockSpec((HH, HI), lambda b, c: (0, 0)),
            pl.BlockSpec((HI, HH), lambda b, c: (0, 0)),
        ],
        out_specs=pl.BlockSpec((1, 1, HH, WW), lambda b, c: (b, c, 0, 0)),
        out_shape=jax.ShapeDtypeStruct((BB, 3, HH, WW), f32),
    )(img, ry, rxt)

    colors5 = colors.reshape(BB, 3, NSUB, RPT, WW)
    seg4 = segments.astype(jnp.int32).reshape(BB, HH, 4, HI)
    xT = x.reshape(BB, CC, N).transpose(0, 2, 1).reshape(BB * N, CC)

    mesh = plsc.VectorSubcoreMesh(core_axis_name="c", subcore_axis_name="s",
                                  num_cores=2, num_subcores=NSUB)
    sc_ab = pl.kernel(
        _sc_ab_body,
        out_type=[
            jax.ShapeDtypeStruct((BB, K, HID), f32),
            jax.ShapeDtypeStruct((BB, 1, K), f32),
            jax.ShapeDtypeStruct((BB, NSUB, RPT, WW), f32),
            jax.ShapeDtypeStruct((BB, K * K), f32),
        ],
        mesh=mesh,
        compiler_params=pltpu.CompilerParams(needs_layout_passes=False,
                                             use_tc_tiling_on_sc=False),
        scratch_types=[
            pltpu.VMEM_SHARED((K,), f32),
            pltpu.VMEM_SHARED((K,), f32),
            pltpu.VMEM_SHARED((K,), f32),
            pltpu.VMEM_SHARED((K,), f32),
            pltpu.VMEM_SHARED((K,), f32),
            pltpu.VMEM_SHARED((K, HID), f32),
            pltpu.VMEM_SHARED((K * K,), f32),
            pltpu.VMEM((RPT + 1, 96), jnp.int32),
            pltpu.VMEM((3, RPT, 96), f32),
            pltpu.VMEM((96,), f32),
            pltpu.VMEM((K,), f32),
            pltpu.VMEM((K,), f32),
            pltpu.VMEM((K,), f32),
            pltpu.VMEM((K,), f32),
            pltpu.VMEM((K,), f32),
            pltpu.VMEM((RPT, 96), f32),
            pltpu.VMEM((RPT, 96), f32),
            pltpu.VMEM((96, HID), f32),
            pltpu.VMEM((96, HID), f32),
            pltpu.VMEM((2 * RPT, 2, 96), jnp.int32),
            pltpu.VMEM((2 * RPT, 96), f32),
            pltpu.VMEM((8192,), f32),
            pltpu.VMEM((64, HID), f32),
            pltpu.VMEM((RPT + 1, HI), jnp.int32),
            pltpu.SemaphoreType.DMA,
            pltpu.SemaphoreType.DMA,
            pltpu.SemaphoreType.DMA,
            pltpu.SemaphoreType.DMA,
            pltpu.SemaphoreType.DMA,
            pltpu.SemaphoreType.DMA,
            pltpu.SemaphoreType.DMA,
        ],
    )
    node_pre, validf, sim3, adjflat = sc_ab(colors5, seg4, xT)
    adjcnt = adjflat.reshape(BB, K, K)

    wfull = lambda s: pl.BlockSpec(s, lambda i: tuple(0 for _ in s))

    def dense_call(bi):
        def bidx3(shape):
            return pl.BlockSpec(shape, lambda i: (bi, 0, 0))
        return pl.pallas_call(
            _dense_body,
            grid=(1,),
            in_specs=[
                bidx3((1, K, HID)),
                bidx3((1, 1, K)),
                bidx3((1, K, K)),
                wfull((HID, CC)), wfull((1, HID)),
                wfull((HID, HID)), wfull((1, HID)),
                wfull((3 * HID, HID)), wfull((1, 3 * HID)),
                wfull((HID, HID)), wfull((1, HID)),
                wfull((2 * HID, HID)), wfull((1, 2 * HID)),
                wfull((HID, 2 * HID)), wfull((1, HID)),
                wfull((1, HID)), wfull((1, HID)),
                wfull((1, HID)), wfull((1, HID)),
            ],
            out_specs=pl.BlockSpec((1, K, HID), lambda i: (0, 0, 0)),
            out_shape=jax.ShapeDtypeStruct((1, K, HID), f32),
        )(node_pre, validf, adjcnt,
          proj_in_w, proj_in_b.reshape(1, -1), gcn_w, gcn_b.reshape(1, -1),
          attn_in_w, attn_in_b.reshape(1, -1), attn_out_w,
          attn_out_b.reshape(1, -1), ff1_w, ff1_b.reshape(1, -1), ff2_w,
          ff2_b.reshape(1, -1), ln1_g.reshape(1, -1), ln1_b.reshape(1, -1),
          ln2_g.reshape(1, -1), ln2_b.reshape(1, -1))

    sim32 = sim3.reshape(BB, 2 * NSUB, RPO, WW)

    def out_call(bi, h_b):
        body = functools.partial(_sc_out_body, bi)
        return pl.kernel(
            body,
            out_type=jax.ShapeDtypeStruct((N, HID), f32),
            mesh=plsc.VectorSubcoreMesh(core_axis_name="c",
                                        subcore_axis_name="s",
                                        num_cores=2, num_subcores=NSUB),
            compiler_params=pltpu.CompilerParams(needs_layout_passes=False,
                                                 use_tc_tiling_on_sc=False),
            scratch_types=[
                pltpu.VMEM((RPO, 96), jnp.int32),
                pltpu.VMEM((RPO, 96), f32),
                pltpu.VMEM((96, HID), f32),
                pltpu.VMEM((96, HID), f32),
                pltpu.VMEM((RPO, HI), jnp.int32),
                pltpu.SemaphoreType.DMA,
                pltpu.SemaphoreType.DMA,
                pltpu.SemaphoreType.DMA,
            ],
        )(h_b, sim32, seg4)

    h0 = dense_call(0)
    h1 = dense_call(1)
    o0 = out_call(0, h0.reshape(K, HID))
    o1 = out_call(1, h1.reshape(K, HID))
    out = jnp.stack([o0, o1], axis=0)
    return out.reshape(BB, N, HID).transpose(0, 2, 1).reshape(BB, HID, HH, WW)


# speedup vs baseline: 1.0504x; 1.0504x over previous
import functools

import numpy as np
import jax
import jax.numpy as jnp
from jax import lax
from jax.experimental import pallas as pl
from jax.experimental.pallas import tpu as pltpu
from jax.experimental.pallas import tpu_sc as plsc

BB, CC, HH, WW = 2, 128, 96, 96
N = HH * WW
K = 1024
HI = 384
HID = 128
NSUB = 16
RPT = HH // NSUB


def _interp_matrix():
    ys = np.linspace(0.0, HI - 1.0, HH)
    y0 = np.clip(np.floor(ys).astype(np.int32), 0, HI - 1)
    y1 = np.clip(y0 + 1, 0, HI - 1)
    wy = (ys - y0).astype(np.float32)
    ry = np.zeros((HH, HI), np.float32)
    ar = np.arange(HH)
    np.add.at(ry, (ar, y0), 1.0 - wy)
    np.add.at(ry, (ar, y1), wy)
    return ry


def _bilinear_body(img_ref, ry_ref, rxt_ref, out_ref):
    ch = img_ref[0, 0]
    e = jnp.dot(ry_ref[...], ch, preferred_element_type=jnp.float32)
    out_ref[0, 0] = jnp.dot(e, rxt_ref[...],
                            preferred_element_type=jnp.float32)


def _rsqrt_vec(s):
    i = plsc.bitcast(s, jnp.int32)
    y = plsc.bitcast(jnp.int32(0x5F3759DF) - (i >> 1), jnp.float32)
    for _ in range(3):
        y = y * (1.5 - 0.5 * s * y * y)
    return y


def _fill1d(ref, n, val):
    v = jnp.full((16,), val, ref.dtype)

    def bd(i, c):
        for q in range(4):
            ref[pl.ds((i * 4 + q) * 16, 16)] = v
        return c

    lax.fori_loop(0, n // 64, bd, 0)


def _fill2d(ref, rows):
    v = jnp.zeros((16,), ref.dtype)

    def bd(i, c):
        for q in range(8):
            ref[i, pl.ds(q * 16, 16)] = v
        return c

    lax.fori_loop(0, rows, bd, 0)


def _scale_rows(ref, sref, srow, nrows):
    def bd(g, c):
        sv = sref[srow, pl.ds(g * 16, 16)]
        for jj in range(16):
            s = sv[jj]
            i = g * 16 + jj
            for q in range(8):
                sl = pl.ds(q * 16, 16)
                ref[i, sl] = ref[i, sl] * s
        return c

    lax.fori_loop(0, nrows // 16, bd, 0)


def _sc_ab_body(colors, seg4, xT,
                node_pre, validf, simout, adjcnt,
                sh_sr, sh_sg, sh_sb, sh_cnt, sh_ws, sh_node, sh_adj,
                v_cid, v_col, v_ones, v_mr, v_mg, v_mb, v_cnt, v_valid,
                v_sim, v_wsq, v_feat0, v_feat1, v_aidx, v_aval, v_z1, v_b64,
                v_segrow, sem_z, sem_a, sem_in, sem_sc, sem_f0, sem_f1,
                sem_d):
    t = lax.axis_index("s")
    b = lax.axis_index("c")
    r0 = t * RPT
    iota16 = lax.iota(jnp.int32, 16)
    feat_bufs = (v_feat0, v_feat1)

    with jax.named_scope("ab_zero"):
        _fill1d(v_z1, 8192, 0.0)
        _fill2d(v_b64, 64)
        one = jnp.full((16,), 1.0, jnp.float32)
        for q in range(6):
            v_ones[pl.ds(q * 16, 16)] = one

        zd = []
        for j in range(8):
            zd.append(pltpu.async_copy(
                v_z1, sh_adj.at[pl.ds(t * 65536 + j * 8192, 8192)], sem_z))
        zd.append(pltpu.async_copy(v_b64, sh_node.at[pl.ds(t * 64, 64)],
                                   sem_z))

        @pl.when(t == 0)
        def _():
            for sh in (sh_sr, sh_sg, sh_sb, sh_cnt, sh_ws):
                pltpu.sync_copy(v_z1.at[pl.ds(0, K)], sh)

        for d in zd:
            d.wait()
        plsc.subcore_barrier()

    with jax.named_scope("ab_stage_a"):
        sd = []
        for r in range(RPT):
            sd.append(pltpu.async_copy(seg4.at[b, r0 + r, 0],
                                       v_segrow.at[r], sem_in))
        for c in range(3):
            sd.append(pltpu.async_copy(colors.at[b, c, t], v_col.at[c],
                                       sem_in))

        @pl.when(t < NSUB - 1)
        def _():
            pltpu.sync_copy(seg4.at[b, (t + 1) * RPT, 0], v_segrow.at[RPT])

        @pl.when(t == NSUB - 1)
        def _():
            z = jnp.zeros((16,), jnp.int32)
            for u in range(6):
                v_cid[RPT, pl.ds(u * 16, 16)] = z

        for d in sd:
            d.wait()
        for r in range(RPT):
            for u in range(6):
                cv = plsc.load_gather(v_segrow.at[r], [iota16 * 4 + u * 64])
                v_cid[r, pl.ds(u * 16, 16)] = cv

        @pl.when(t < NSUB - 1)
        def _():
            for u in range(6):
                cv = plsc.load_gather(v_segrow.at[RPT], [iota16 * 4 + u * 64])
                v_cid[RPT, pl.ds(u * 16, 16)] = cv
        ad = []
        for r in range(RPT):
            idx = v_cid.at[r]
            ad.append(pltpu.async_copy(v_col.at[0, r], sh_sr.at[idx], sem_a,
                                       add=True))
            ad.append(pltpu.async_copy(v_col.at[1, r], sh_sg.at[idx], sem_a,
                                       add=True))
            ad.append(pltpu.async_copy(v_col.at[2, r], sh_sb.at[idx], sem_a,
                                       add=True))
            ad.append(pltpu.async_copy(v_ones, sh_cnt.at[idx], sem_a,
                                       add=True))
        for d in ad:
            d.wait()
        plsc.subcore_barrier()

    with jax.named_scope("ab_means"):
        pltpu.sync_copy(sh_sr, v_mr)
        pltpu.sync_copy(sh_sg, v_mg)
        pltpu.sync_copy(sh_sb, v_mb)
        pltpu.sync_copy(sh_cnt, v_cnt)

        def mean_bd(i, c):
            sl = pl.ds(i * 16, 16)
            cnt = v_cnt[sl]
            rc = 1.0 / (cnt + 1e-8)
            v_mr[sl] = v_mr[sl] * rc
            v_mg[sl] = v_mg[sl] * rc
            v_mb[sl] = v_mb[sl] * rc
            v_valid[sl] = jnp.where(cnt > 0.0, 1.0, 0.0)
            return c

        lax.fori_loop(0, K // 16, mean_bd, 0)

        @pl.when(t == 0)
        def _():
            pltpu.sync_copy(v_valid, validf.at[b, 0])

    with jax.named_scope("ab_rows"):
        feat_in = [None, None]
        feat_sc = [None, None]
        feat_in[0] = pltpu.async_copy(xT.at[pl.ds(b * N + r0 * 96, 96)],
                                      feat_bufs[0], sem_in)
        row_descs = [None] * RPT
        for r in range(RPT):
            gr = r0 + r
            p = r % 2
            rowfull = jnp.full((16,), r, jnp.int32)
            feat_in[p].wait()
            if r + 1 < RPT:
                if feat_sc[1 - p] is not None:
                    feat_sc[1 - p].wait()
                feat_in[1 - p] = pltpu.async_copy(
                    xT.at[pl.ds(b * N + (gr + 1) * 96, 96)],
                    feat_bufs[1 - p], sem_in)
            for u in range(6):
                sl = pl.ds(u * 16, 16)
                cidv = v_cid[r, sl]
                cr = v_col[0, r, sl]
                cg = v_col[1, r, sl]
                cb = v_col[2, r, sl]
                mr = plsc.load_gather(v_mr, [cidv])
                mg = plsc.load_gather(v_mg, [cidv])
                mb = plsc.load_gather(v_mb, [cidv])
                dot = cr * mr + cg * mg + cb * mb
                n1 = jnp.maximum(cr * cr + cg * cg + cb * cb, 1e-16)
                n2 = jnp.maximum(mr * mr + mg * mg + mb * mb, 1e-16)
                sim = jnp.clip(dot * _rsqrt_vec(n1) * _rsqrt_vec(n2), 0.0, 1.0)
                v_sim[r, sl] = sim
                v_wsq[r, sl] = sim * sim
                j = iota16 + (u * 16)
                jn = jnp.minimum(j + 1, 95)
                bh = plsc.load_gather(v_cid, [rowfull, jn])
                okh = jnp.logical_and(cidv != bh, j < 95)
                v_aval[2 * r, sl] = jnp.where(okh, 1.0, 0.0)
                v_aidx[2 * r, 0, sl] = cidv * K + bh
                v_aidx[2 * r, 1, sl] = bh * K + cidv
                bv = v_cid[r + 1, sl]
                okv = jnp.logical_and(cidv != bv, gr < HH - 1)
                v_aval[2 * r + 1, sl] = jnp.where(okv, 1.0, 0.0)
                v_aidx[2 * r + 1, 0, sl] = cidv * K + bv
                v_aidx[2 * r + 1, 1, sl] = bv * K + cidv
            rd = []
            for hv in range(2):
                for d2 in range(2):
                    rd.append(pltpu.async_copy(
                        v_aval.at[2 * r + hv],
                        sh_adj.at[v_aidx.at[2 * r + hv, d2]], sem_sc,
                        add=True))
            rd.append(pltpu.async_copy(v_wsq.at[r], sh_ws.at[v_cid.at[r]],
                                       sem_sc, add=True))
            row_descs[r] = rd
            _scale_rows(feat_bufs[p], v_sim, r, 96)
            feat_sc[p] = pltpu.async_copy(feat_bufs[p],
                                          sh_node.at[v_cid.at[r]],
                                          (sem_f0, sem_f1)[p], add=True)
            if r >= 1:
                for d in row_descs[r - 1]:
                    d.wait()
        for d in row_descs[RPT - 1]:
            d.wait()
        for p in range(2):
            if feat_sc[p] is not None:
                feat_sc[p].wait()
        pltpu.sync_copy(v_sim, simout.at[b, t])
        plsc.subcore_barrier()

    with jax.named_scope("ab_dump"):
        adj_d = pltpu.async_copy(sh_adj.at[pl.ds(t * 65536, 65536)],
                                 adjcnt.at[b, pl.ds(t * 65536, 65536)], sem_d)
        pltpu.sync_copy(sh_node.at[pl.ds(t * 64, 64)], v_b64)
        pltpu.sync_copy(sh_ws.at[pl.ds(t * 64, 64)], v_wsq.at[0, pl.ds(0, 64)])

        def div_bd(g, c):
            wv = v_wsq[0, pl.ds(g * 16, 16)]
            rcv = 1.0 / (wv + 1e-8)
            for jj in range(16):
                rc = rcv[jj]
                i = g * 16 + jj
                for q in range(8):
                    sl = pl.ds(q * 16, 16)
                    v_b64[i, sl] = v_b64[i, sl] * rc
            return c

        lax.fori_loop(0, 4, div_bd, 0)
        pltpu.sync_copy(v_b64, node_pre.at[b, pl.ds(t * 64, 64)])
        adj_d.wait()


def _ln_rows(xv, g, bvec):
    m = jnp.mean(xv, axis=-1, keepdims=True)
    d = xv - m
    v = jnp.mean(d * d, axis=-1, keepdims=True)
    return d * lax.rsqrt(v + 1e-5) * g + bvec


def _dense_body(np_ref, va_ref, adj_ref, pw, pb, gw, gb, aw, ab, aow, aob,
                f1w, f1b, f2w, f2b, l1g, l1b, l2g, l2b, out_ref):
    npre = np_ref[0]
    adjc = adj_ref[0]
    validr = va_ref[0]

    def matr(xv, wref):
        return lax.dot_general(xv, wref[...], (((1,), (1,)), ((), ())),
                               preferred_element_type=jnp.float32)

    node = matr(npre, pw) + pb[...]
    adj = (adjc > 0.0).astype(jnp.float32)
    deg = jnp.sum(adj, axis=1, keepdims=True) + 1.0
    dinv = lax.rsqrt(jnp.maximum(deg, 1e-12))
    xw = matr(node, gw)
    y = dinv * xw
    ay = lax.dot_general(adj, y, (((1,), (0,)), ((), ())),
                         preferred_element_type=jnp.float32)
    node = jax.nn.relu(dinv * (ay + y) + gb[...])

    qkv = matr(node, aw) + ab[...]
    heads = []
    for hh in range(4):
        q = qkv[:, 32 * hh:32 * hh + 32]
        kk = qkv[:, 128 + 32 * hh:128 + 32 * hh + 32]
        vv = qkv[:, 256 + 32 * hh:256 + 32 * hh + 32]
        lg = lax.dot_general(q, kk, (((1,), (1,)), ((), ())),
                             preferred_element_type=jnp.float32)
        lg = lg * (1.0 / np.sqrt(32.0))
        lg = jnp.where(validr > 0.0, lg, -1e30)
        mx = jnp.max(lg, axis=-1, keepdims=True)
        e = jnp.exp(lg - mx)
        s = jnp.sum(e, axis=-1, keepdims=True)
        o_h = lax.dot_general(e, vv, (((1,), (0,)), ((), ())),
                              preferred_element_type=jnp.float32) / s
        heads.append(o_h)
    o = jnp.concatenate(heads, axis=1)
    att = matr(o, aow) + aob[...]
    h1 = _ln_rows(node + att, l1g[...], l1b[...])
    ff = matr(jax.nn.relu(matr(h1, f1w) + f1b[...]), f2w) + f2b[...]
    out_ref[0] = _ln_rows(h1 + ff, l2g[...], l2b[...])


def _sc_out_body(h2d, sim3, seg4, outT,
                 v_idx, v_sim, v_rows0, v_rows1, v_segrow, sem_g, sem_o0,
                 sem_o1):
    t = lax.axis_index("s")
    b = lax.axis_index("c")
    r0 = t * RPT
    iota16 = lax.iota(jnp.int32, 16)
    bufs = (v_rows0, v_rows1)
    sd = [pltpu.async_copy(sim3.at[b, t], v_sim, sem_g)]
    for r in range(RPT):
        sd.append(pltpu.async_copy(seg4.at[b, r0 + r, 0], v_segrow.at[r],
                                   sem_g))
    for d in sd:
        d.wait()
    koff = b * K
    for r in range(RPT):
        for u in range(6):
            cv = plsc.load_gather(v_segrow.at[r], [iota16 * 4 + u * 64])
            v_idx[r, pl.ds(u * 16, 16)] = cv + koff
    gin = [None, None]
    gout = [None, None]
    gin[0] = pltpu.async_copy(h2d.at[v_idx.at[0]], bufs[0], sem_g)
    for r in range(RPT):
        p = r % 2
        gin[p].wait()
        if r + 1 < RPT:
            if gout[1 - p] is not None:
                gout[1 - p].wait()
            gin[1 - p] = pltpu.async_copy(h2d.at[v_idx.at[r + 1]],
                                          bufs[1 - p], sem_g)
        _scale_rows(bufs[p], v_sim, r, 96)
        gout[p] = pltpu.async_copy(
            bufs[p], outT.at[pl.ds(b * N + (r0 + r) * 96, 96)],
            (sem_o0, sem_o1)[p])
    for p in range(2):
        if gout[p] is not None:
            gout[p].wait()


def kernel(x, img, segments, proj_in_w, proj_in_b, gcn_w, gcn_b, attn_in_w,
           attn_in_b, attn_out_w, attn_out_b, ff1_w, ff1_b, ff2_w, ff2_b,
           ln1_g, ln1_b, ln2_g, ln2_b):
    f32 = jnp.float32
    ry = jnp.asarray(_interp_matrix())
    rxt = jnp.asarray(_interp_matrix().T)

    colors = pl.pallas_call(
        _bilinear_body,
        grid=(BB, 3),
        in_specs=[
            pl.BlockSpec((1, 1, HI, HI), lambda b, c: (b, c, 0, 0)),
            pl.BlockSpec((HH, HI), lambda b, c: (0, 0)),
            pl.BlockSpec((HI, HH), lambda b, c: (0, 0)),
        ],
        out_specs=pl.BlockSpec((1, 1, HH, WW), lambda b, c: (b, c, 0, 0)),
        out_shape=jax.ShapeDtypeStruct((BB, 3, HH, WW), f32),
    )(img, ry, rxt)

    colors5 = colors.reshape(BB, 3, NSUB, RPT, WW)
    seg4 = segments.astype(jnp.int32).reshape(BB, HH, 4, HI)
    xT = x.reshape(BB, CC, N).transpose(0, 2, 1).reshape(BB * N, CC)

    mesh = plsc.VectorSubcoreMesh(core_axis_name="c", subcore_axis_name="s",
                                  num_cores=2, num_subcores=NSUB)
    sc_ab = pl.kernel(
        _sc_ab_body,
        out_type=[
            jax.ShapeDtypeStruct((BB, K, HID), f32),
            jax.ShapeDtypeStruct((BB, 1, K), f32),
            jax.ShapeDtypeStruct((BB, NSUB, RPT, WW), f32),
            jax.ShapeDtypeStruct((BB, K * K), f32),
        ],
        mesh=mesh,
        compiler_params=pltpu.CompilerParams(needs_layout_passes=False,
                                             use_tc_tiling_on_sc=False),
        scratch_types=[
            pltpu.VMEM_SHARED((K,), f32),
            pltpu.VMEM_SHARED((K,), f32),
            pltpu.VMEM_SHARED((K,), f32),
            pltpu.VMEM_SHARED((K,), f32),
            pltpu.VMEM_SHARED((K,), f32),
            pltpu.VMEM_SHARED((K, HID), f32),
            pltpu.VMEM_SHARED((K * K,), f32),
            pltpu.VMEM((RPT + 1, 96), jnp.int32),
            pltpu.VMEM((3, RPT, 96), f32),
            pltpu.VMEM((96,), f32),
            pltpu.VMEM((K,), f32),
            pltpu.VMEM((K,), f32),
            pltpu.VMEM((K,), f32),
            pltpu.VMEM((K,), f32),
            pltpu.VMEM((K,), f32),
            pltpu.VMEM((RPT, 96), f32),
            pltpu.VMEM((RPT, 96), f32),
            pltpu.VMEM((96, HID), f32),
            pltpu.VMEM((96, HID), f32),
            pltpu.VMEM((2 * RPT, 2, 96), jnp.int32),
            pltpu.VMEM((2 * RPT, 96), f32),
            pltpu.VMEM((8192,), f32),
            pltpu.VMEM((64, HID), f32),
            pltpu.VMEM((RPT + 1, HI), jnp.int32),
            pltpu.SemaphoreType.DMA,
            pltpu.SemaphoreType.DMA,
            pltpu.SemaphoreType.DMA,
            pltpu.SemaphoreType.DMA,
            pltpu.SemaphoreType.DMA,
            pltpu.SemaphoreType.DMA,
            pltpu.SemaphoreType.DMA,
        ],
    )
    node_pre, validf, sim3, adjflat = sc_ab(colors5, seg4, xT)
    adjcnt = adjflat.reshape(BB, K, K)

    wfull = lambda s: pl.BlockSpec(s, lambda i: tuple(0 for _ in s))
    h = pl.pallas_call(
        _dense_body,
        grid=(BB,),
        in_specs=[
            pl.BlockSpec((1, K, HID), lambda i: (i, 0, 0)),
            pl.BlockSpec((1, 1, K), lambda i: (i, 0, 0)),
            pl.BlockSpec((1, K, K), lambda i: (i, 0, 0)),
            wfull((HID, CC)), wfull((1, HID)),
            wfull((HID, HID)), wfull((1, HID)),
            wfull((3 * HID, HID)), wfull((1, 3 * HID)),
            wfull((HID, HID)), wfull((1, HID)),
            wfull((2 * HID, HID)), wfull((1, 2 * HID)),
            wfull((HID, 2 * HID)), wfull((1, HID)),
            wfull((1, HID)), wfull((1, HID)),
            wfull((1, HID)), wfull((1, HID)),
        ],
        out_specs=pl.BlockSpec((1, K, HID), lambda i: (i, 0, 0)),
        out_shape=jax.ShapeDtypeStruct((BB, K, HID), f32),
    )(node_pre, validf, adjcnt,
      proj_in_w, proj_in_b.reshape(1, -1), gcn_w, gcn_b.reshape(1, -1),
      attn_in_w, attn_in_b.reshape(1, -1), attn_out_w,
      attn_out_b.reshape(1, -1), ff1_w, ff1_b.reshape(1, -1), ff2_w,
      ff2_b.reshape(1, -1), ln1_g.reshape(1, -1), ln1_b.reshape(1, -1),
      ln2_g.reshape(1, -1), ln2_b.reshape(1, -1))

    sc_out = pl.kernel(
        _sc_out_body,
        out_type=jax.ShapeDtypeStruct((BB * N, HID), f32),
        mesh=plsc.VectorSubcoreMesh(core_axis_name="c", subcore_axis_name="s",
                                    num_cores=2, num_subcores=NSUB),
        compiler_params=pltpu.CompilerParams(needs_layout_passes=False,
                                             use_tc_tiling_on_sc=False),
        scratch_types=[
            pltpu.VMEM((RPT, 96), jnp.int32),
            pltpu.VMEM((RPT, 96), f32),
            pltpu.VMEM((96, HID), f32),
            pltpu.VMEM((96, HID), f32),
            pltpu.VMEM((RPT, HI), jnp.int32),
            pltpu.SemaphoreType.DMA,
            pltpu.SemaphoreType.DMA,
            pltpu.SemaphoreType.DMA,
        ],
    )
    outT = sc_out(h.reshape(BB * K, HID), sim3, seg4)
    return outT.reshape(BB, N, HID).transpose(0, 2, 1).reshape(BB, HID, HH, WW)


# speedup vs baseline: 1.0505x; 1.0001x over previous
import numpy as np
import jax
import jax.numpy as jnp
from jax import lax
from jax.experimental import pallas as pl
from jax.experimental.pallas import tpu as pltpu
from jax.experimental.pallas import tpu_sc as plsc

BB, CC, HH, WW = 2, 128, 96, 96
N = HH * WW
K = 1024
HI = 384
HID = 128
NSUB = 16
RPT = HH // NSUB


def _interp_matrix():
    ys = np.linspace(0.0, HI - 1.0, HH)
    y0 = np.clip(np.floor(ys).astype(np.int32), 0, HI - 1)
    y1 = np.clip(y0 + 1, 0, HI - 1)
    wy = (ys - y0).astype(np.float32)
    ry = np.zeros((HH, HI), np.float32)
    ar = np.arange(HH)
    np.add.at(ry, (ar, y0), 1.0 - wy)
    np.add.at(ry, (ar, y1), wy)
    return ry


def _bilinear_body(img_ref, ry_ref, rxt_ref, out_ref):
    ch = img_ref[0, 0]
    e = jnp.dot(ry_ref[...], ch, preferred_element_type=jnp.float32)
    out_ref[0, 0] = jnp.dot(e, rxt_ref[...],
                            preferred_element_type=jnp.float32)


def _rsqrt_vec(s):
    i = plsc.bitcast(s, jnp.int32)
    y = plsc.bitcast(jnp.int32(0x5F3759DF) - (i >> 1), jnp.float32)
    for _ in range(3):
        y = y * (1.5 - 0.5 * s * y * y)
    return y


def _fill1d(ref, n, val):
    v = jnp.full((16,), val, ref.dtype)

    def bd(i, c):
        for q in range(4):
            ref[pl.ds((i * 4 + q) * 16, 16)] = v
        return c

    lax.fori_loop(0, n // 64, bd, 0)


def _fill2d(ref, rows):
    v = jnp.zeros((16,), ref.dtype)

    def bd(i, c):
        for q in range(8):
            ref[i, pl.ds(q * 16, 16)] = v
        return c

    lax.fori_loop(0, rows, bd, 0)


def _scale_rows(ref, sref, srow, nrows):
    def bd(g, c):
        sv = sref[srow, pl.ds(g * 16, 16)]
        for jj in range(16):
            s = sv[jj]
            i = g * 16 + jj
            for q in range(8):
                sl = pl.ds(q * 16, 16)
                ref[i, sl] = ref[i, sl] * s
        return c

    lax.fori_loop(0, nrows // 16, bd, 0)


def _sc_ab_body(colors, seg4, xT,
                node_pre, validf, simout, adjcnt,
                sh_sr, sh_sg, sh_sb, sh_cnt, sh_ws, sh_node, sh_adj,
                v_cid, v_col, v_ones, v_mr, v_mg, v_mb, v_cnt, v_valid,
                v_sim, v_wsq, v_feat0, v_feat1, v_aidx, v_aval, v_z1, v_b64,
                v_segrow, sem_z, sem_a, sem_in, sem_sc, sem_f0, sem_f1,
                sem_d):
    t = lax.axis_index("s")
    b = lax.axis_index("c")
    r0 = t * RPT
    iota16 = lax.iota(jnp.int32, 16)
    feat_bufs = (v_feat0, v_feat1)

    with jax.named_scope("ab_zero"):
        _fill1d(v_z1, 8192, 0.0)
        _fill2d(v_b64, 64)
        one = jnp.full((16,), 1.0, jnp.float32)
        for q in range(6):
            v_ones[pl.ds(q * 16, 16)] = one

        zd = []
        for j in range(8):
            zd.append(pltpu.async_copy(
                v_z1, sh_adj.at[pl.ds(t * 65536 + j * 8192, 8192)], sem_z))
        zd.append(pltpu.async_copy(v_b64, sh_node.at[pl.ds(t * 64, 64)],
                                   sem_z))

        @pl.when(t == 0)
        def _():
            for sh in (sh_sr, sh_sg, sh_sb, sh_cnt, sh_ws):
                pltpu.sync_copy(v_z1.at[pl.ds(0, K)], sh)

        for d in zd:
            d.wait()
        plsc.subcore_barrier()

    with jax.named_scope("ab_stage_a"):
        sd = []
        for r in range(RPT):
            sd.append(pltpu.async_copy(seg4.at[b, r0 + r, 0],
                                       v_segrow.at[r], sem_in))
        for c in range(3):
            sd.append(pltpu.async_copy(colors.at[b, c, t], v_col.at[c],
                                       sem_in))

        @pl.when(t < NSUB - 1)
        def _():
            pltpu.sync_copy(seg4.at[b, (t + 1) * RPT, 0], v_segrow.at[RPT])

        @pl.when(t == NSUB - 1)
        def _():
            z = jnp.zeros((16,), jnp.int32)
            for u in range(6):
                v_cid[RPT, pl.ds(u * 16, 16)] = z

        for d in sd:
            d.wait()
        for r in range(RPT):
            for u in range(6):
                cv = plsc.load_gather(v_segrow.at[r], [iota16 * 4 + u * 64])
                v_cid[r, pl.ds(u * 16, 16)] = cv

        @pl.when(t < NSUB - 1)
        def _():
            for u in range(6):
                cv = plsc.load_gather(v_segrow.at[RPT], [iota16 * 4 + u * 64])
                v_cid[RPT, pl.ds(u * 16, 16)] = cv
        ad = []
        for r in range(RPT):
            idx = v_cid.at[r]
            ad.append(pltpu.async_copy(v_col.at[0, r], sh_sr.at[idx], sem_a,
                                       add=True))
            ad.append(pltpu.async_copy(v_col.at[1, r], sh_sg.at[idx], sem_a,
                                       add=True))
            ad.append(pltpu.async_copy(v_col.at[2, r], sh_sb.at[idx], sem_a,
                                       add=True))
            ad.append(pltpu.async_copy(v_ones, sh_cnt.at[idx], sem_a,
                                       add=True))
        for d in ad:
            d.wait()
        plsc.subcore_barrier()

    with jax.named_scope("ab_means"):
        pltpu.sync_copy(sh_sr, v_mr)
        pltpu.sync_copy(sh_sg, v_mg)
        pltpu.sync_copy(sh_sb, v_mb)
        pltpu.sync_copy(sh_cnt, v_cnt)

        def mean_bd(i, c):
            sl = pl.ds(i * 16, 16)
            cnt = v_cnt[sl]
            rc = 1.0 / (cnt + 1e-8)
            v_mr[sl] = v_mr[sl] * rc
            v_mg[sl] = v_mg[sl] * rc
            v_mb[sl] = v_mb[sl] * rc
            v_valid[sl] = jnp.where(cnt > 0.0, 1.0, 0.0)
            return c

        lax.fori_loop(0, K // 16, mean_bd, 0)

        @pl.when(t == 0)
        def _():
            pltpu.sync_copy(v_valid, validf.at[b, 0])

    with jax.named_scope("ab_rows"):
        feat_in = [None, None]
        feat_sc = [None, None]
        feat_in[0] = pltpu.async_copy(xT.at[pl.ds(b * N + r0 * 96, 96)],
                                      feat_bufs[0], sem_in)
        row_descs = [None] * RPT
        for r in range(RPT):
            gr = r0 + r
            p = r % 2
            rowfull = jnp.full((16,), r, jnp.int32)
            feat_in[p].wait()
            if r + 1 < RPT:
                if feat_sc[1 - p] is not None:
                    feat_sc[1 - p].wait()
                feat_in[1 - p] = pltpu.async_copy(
                    xT.at[pl.ds(b * N + (gr + 1) * 96, 96)],
                    feat_bufs[1 - p], sem_in)
            for u in range(6):
                sl = pl.ds(u * 16, 16)
                cidv = v_cid[r, sl]
                cr = v_col[0, r, sl]
                cg = v_col[1, r, sl]
                cb = v_col[2, r, sl]
                mr = plsc.load_gather(v_mr, [cidv])
                mg = plsc.load_gather(v_mg, [cidv])
                mb = plsc.load_gather(v_mb, [cidv])
                dot = cr * mr + cg * mg + cb * mb
                n1 = jnp.maximum(cr * cr + cg * cg + cb * cb, 1e-16)
                n2 = jnp.maximum(mr * mr + mg * mg + mb * mb, 1e-16)
                sim = jnp.clip(dot * _rsqrt_vec(n1) * _rsqrt_vec(n2), 0.0, 1.0)
                v_sim[r, sl] = sim
                v_wsq[r, sl] = sim * sim
                j = iota16 + (u * 16)
                jn = jnp.minimum(j + 1, 95)
                bh = plsc.load_gather(v_cid, [rowfull, jn])
                okh = jnp.logical_and(cidv != bh, j < 95)
                v_aval[2 * r, sl] = jnp.where(okh, 1.0, 0.0)
                v_aidx[2 * r, 0, sl] = cidv * K + bh
                v_aidx[2 * r, 1, sl] = bh * K + cidv
                bv = v_cid[r + 1, sl]
                okv = jnp.logical_and(cidv != bv, gr < HH - 1)
                v_aval[2 * r + 1, sl] = jnp.where(okv, 1.0, 0.0)
                v_aidx[2 * r + 1, 0, sl] = cidv * K + bv
                v_aidx[2 * r + 1, 1, sl] = bv * K + cidv
            rd = []
            for hv in range(2):
                for d2 in range(2):
                    rd.append(pltpu.async_copy(
                        v_aval.at[2 * r + hv],
                        sh_adj.at[v_aidx.at[2 * r + hv, d2]], sem_sc,
                        add=True))
            rd.append(pltpu.async_copy(v_wsq.at[r], sh_ws.at[v_cid.at[r]],
                                       sem_sc, add=True))
            row_descs[r] = rd
            _scale_rows(feat_bufs[p], v_sim, r, 96)
            feat_sc[p] = pltpu.async_copy(feat_bufs[p],
                                          sh_node.at[v_cid.at[r]],
                                          (sem_f0, sem_f1)[p], add=True)
            if r >= 1:
                for d in row_descs[r - 1]:
                    d.wait()
        for d in row_descs[RPT - 1]:
            d.wait()
        for p in range(2):
            if feat_sc[p] is not None:
                feat_sc[p].wait()
        pltpu.sync_copy(v_sim, simout.at[b, t])
        plsc.subcore_barrier()

    with jax.named_scope("ab_dump"):
        adj_d = pltpu.async_copy(sh_adj.at[pl.ds(t * 65536, 65536)],
                                 adjcnt.at[b, pl.ds(t * 65536, 65536)], sem_d)
        pltpu.sync_copy(sh_node.at[pl.ds(t * 64, 64)], v_b64)
        pltpu.sync_copy(sh_ws.at[pl.ds(t * 64, 64)], v_wsq.at[0, pl.ds(0, 64)])

        def div_bd(g, c):
            wv = v_wsq[0, pl.ds(g * 16, 16)]
            rcv = 1.0 / (wv + 1e-8)
            for jj in range(16):
                rc = rcv[jj]
                i = g * 16 + jj
                for q in range(8):
                    sl = pl.ds(q * 16, 16)
                    v_b64[i, sl] = v_b64[i, sl] * rc
            return c

        lax.fori_loop(0, 4, div_bd, 0)
        pltpu.sync_copy(v_b64, node_pre.at[b, pl.ds(t * 64, 64)])
        adj_d.wait()


def _ln_rows(xv, g, bvec):
    m = jnp.mean(xv, axis=-1, keepdims=True)
    d = xv - m
    v = jnp.mean(d * d, axis=-1, keepdims=True)
    return d * lax.rsqrt(v + 1e-5) * g + bvec


def _dense_body(np_ref, va_ref, adj_ref, pw, pb, gw, gb, aw, ab, aow, aob,
                f1w, f1b, f2w, f2b, l1g, l1b, l2g, l2b, out_ref):
    npre = np_ref[0]
    adjc = adj_ref[0]
    validr = va_ref[0]

    def matr(xv, wref):
        return lax.dot_general(xv, wref[...], (((1,), (1,)), ((), ())),
                               preferred_element_type=jnp.float32)

    node = matr(npre, pw) + pb[...]
    adj = (adjc > 0.0).astype(jnp.float32)
    deg = jnp.sum(adj, axis=1, keepdims=True) + 1.0
    dinv = lax.rsqrt(jnp.maximum(deg, 1e-12))
    xw = matr(node, gw)
    y = dinv * xw
    ay = lax.dot_general(adj, y, (((1,), (0,)), ((), ())),
                         preferred_element_type=jnp.float32)
    node = jax.nn.relu(dinv * (ay + y) + gb[...])

    qkv = matr(node, aw) + ab[...]
    heads = []
    for hh in range(4):
        q = qkv[:, 32 * hh:32 * hh + 32]
        kk = qkv[:, 128 + 32 * hh:128 + 32 * hh + 32]
        vv = qkv[:, 256 + 32 * hh:256 + 32 * hh + 32]
        lg = lax.dot_general(q, kk, (((1,), (1,)), ((), ())),
                             preferred_element_type=jnp.float32)
        lg = lg * (1.0 / np.sqrt(32.0))
        lg = jnp.where(validr > 0.0, lg, -1e30)
        mx = jnp.max(lg, axis=-1, keepdims=True)
        e = jnp.exp(lg - mx)
        s = jnp.sum(e, axis=-1, keepdims=True)
        o_h = lax.dot_general(e, vv, (((1,), (0,)), ((), ())),
                              preferred_element_type=jnp.float32) / s
        heads.append(o_h)
    o = jnp.concatenate(heads, axis=1)
    att = matr(o, aow) + aob[...]
    h1 = _ln_rows(node + att, l1g[...], l1b[...])
    ff = matr(jax.nn.relu(matr(h1, f1w) + f1b[...]), f2w) + f2b[...]
    out_ref[0] = _ln_rows(h1 + ff, l2g[...], l2b[...])


def _sc_out_body(h2d, sim3, seg4, outT,
                 v_idx, v_sim, v_rows0, v_rows1, v_segrow, sem_g, sem_o0,
                 sem_o1):
    t = lax.axis_index("s")
    b = lax.axis_index("c")
    r0 = t * RPT
    iota16 = lax.iota(jnp.int32, 16)
    bufs = (v_rows0, v_rows1)
    sd = [pltpu.async_copy(sim3.at[b, t], v_sim, sem_g)]
    for r in range(RPT):
        sd.append(pltpu.async_copy(seg4.at[b, r0 + r, 0], v_segrow.at[r],
                                   sem_g))
    for d in sd:
        d.wait()
    koff = b * K
    for r in range(RPT):
        for u in range(6):
            cv = plsc.load_gather(v_segrow.at[r], [iota16 * 4 + u * 64])
            v_idx[r, pl.ds(u * 16, 16)] = cv + koff
    gin = [None, None]
    gout = [None, None]
    gin[0] = pltpu.async_copy(h2d.at[v_idx.at[0]], bufs[0], sem_g)
    for r in range(RPT):
        p = r % 2
        gin[p].wait()
        if r + 1 < RPT:
            if gout[1 - p] is not None:
                gout[1 - p].wait()
            gin[1 - p] = pltpu.async_copy(h2d.at[v_idx.at[r + 1]],
                                          bufs[1 - p], sem_g)
        _scale_rows(bufs[p], v_sim, r, 96)
        gout[p] = pltpu.async_copy(
            bufs[p], outT.at[pl.ds(b * N + (r0 + r) * 96, 96)],
            (sem_o0, sem_o1)[p])
    for p in range(2):
        if gout[p] is not None:
            gout[p].wait()


def kernel(x, img, segments, proj_in_w, proj_in_b, gcn_w, gcn_b, attn_in_w,
           attn_in_b, attn_out_w, attn_out_b, ff1_w, ff1_b, ff2_w, ff2_b,
           ln1_g, ln1_b, ln2_g, ln2_b):
    f32 = jnp.float32
    ry = jnp.asarray(_interp_matrix())
    rxt = jnp.asarray(_interp_matrix().T)

    colors = pl.pallas_call(
        _bilinear_body,
        grid=(BB, 3),
        in_specs=[
            pl.BlockSpec((1, 1, HI, HI), lambda b, c: (b, c, 0, 0)),
            pl.BlockSpec((HH, HI), lambda b, c: (0, 0)),
            pl.BlockSpec((HI, HH), lambda b, c: (0, 0)),
        ],
        out_specs=pl.BlockSpec((1, 1, HH, WW), lambda b, c: (b, c, 0, 0)),
        out_shape=jax.ShapeDtypeStruct((BB, 3, HH, WW), f32),
    )(img, ry, rxt)

    colors5 = colors.reshape(BB, 3, NSUB, RPT, WW)
    seg4 = segments.astype(jnp.int32).reshape(BB, HH, 4, HI)
    xT = x.reshape(BB, CC, N).transpose(0, 2, 1).reshape(BB * N, CC)

    mesh = plsc.VectorSubcoreMesh(core_axis_name="c", subcore_axis_name="s",
                                  num_cores=2, num_subcores=NSUB)
    sc_ab = pl.kernel(
        _sc_ab_body,
        out_type=[
            jax.ShapeDtypeStruct((BB, K, HID), f32),
            jax.ShapeDtypeStruct((BB, 1, K), f32),
            jax.ShapeDtypeStruct((BB, NSUB, RPT, WW), f32),
            jax.ShapeDtypeStruct((BB, K * K), f32),
        ],
        mesh=mesh,
        compiler_params=pltpu.CompilerParams(needs_layout_passes=False,
                                             use_tc_tiling_on_sc=False),
        scratch_types=[
            pltpu.VMEM_SHARED((K,), f32),
            pltpu.VMEM_SHARED((K,), f32),
            pltpu.VMEM_SHARED((K,), f32),
            pltpu.VMEM_SHARED((K,), f32),
            pltpu.VMEM_SHARED((K,), f32),
            pltpu.VMEM_SHARED((K, HID), f32),
            pltpu.VMEM_SHARED((K * K,), f32),
            pltpu.VMEM((RPT + 1, 96), jnp.int32),
            pltpu.VMEM((3, RPT, 96), f32),
            pltpu.VMEM((96,), f32),
            pltpu.VMEM((K,), f32),
            pltpu.VMEM((K,), f32),
            pltpu.VMEM((K,), f32),
            pltpu.VMEM((K,), f32),
            pltpu.VMEM((K,), f32),
            pltpu.VMEM((RPT, 96), f32),
            pltpu.VMEM((RPT, 96), f32),
            pltpu.VMEM((96, HID), f32),
            pltpu.VMEM((96, HID), f32),
            pltpu.VMEM((2 * RPT, 2, 96), jnp.int32),
            pltpu.VMEM((2 * RPT, 96), f32),
            pltpu.VMEM((8192,), f32),
            pltpu.VMEM((64, HID), f32),
            pltpu.VMEM((RPT + 1, HI), jnp.int32),
            pltpu.SemaphoreType.DMA,
            pltpu.SemaphoreType.DMA,
            pltpu.SemaphoreType.DMA,
            pltpu.SemaphoreType.DMA,
            pltpu.SemaphoreType.DMA,
            pltpu.SemaphoreType.DMA,
            pltpu.SemaphoreType.DMA,
        ],
    )
    node_pre, validf, sim3, adjflat = sc_ab(colors5, seg4, xT)
    adjcnt = adjflat.reshape(BB, K, K)

    wfull = lambda s: pl.BlockSpec(s, lambda i: tuple(0 for _ in s))
    h = pl.pallas_call(
        _dense_body,
        grid=(BB,),
        in_specs=[
            pl.BlockSpec((1, K, HID), lambda i: (i, 0, 0)),
            pl.BlockSpec((1, 1, K), lambda i: (i, 0, 0)),
            pl.BlockSpec((1, K, K), lambda i: (i, 0, 0)),
            wfull((HID, CC)), wfull((1, HID)),
            wfull((HID, HID)), wfull((1, HID)),
            wfull((3 * HID, HID)), wfull((1, 3 * HID)),
            wfull((HID, HID)), wfull((1, HID)),
            wfull((2 * HID, HID)), wfull((1, 2 * HID)),
            wfull((HID, 2 * HID)), wfull((1, HID)),
            wfull((1, HID)), wfull((1, HID)),
            wfull((1, HID)), wfull((1, HID)),
        ],
        out_specs=pl.BlockSpec((1, K, HID), lambda i: (i, 0, 0)),
        out_shape=jax.ShapeDtypeStruct((BB, K, HID), f32),
    )(node_pre, validf, adjcnt,
      proj_in_w, proj_in_b.reshape(1, -1), gcn_w, gcn_b.reshape(1, -1),
      attn_in_w, attn_in_b.reshape(1, -1), attn_out_w,
      attn_out_b.reshape(1, -1), ff1_w, ff1_b.reshape(1, -1), ff2_w,
      ff2_b.reshape(1, -1), ln1_g.reshape(1, -1), ln1_b.reshape(1, -1),
      ln2_g.reshape(1, -1), ln2_b.reshape(1, -1))

    sc_out = pl.kernel(
        _sc_out_body,
        out_type=jax.ShapeDtypeStruct((BB * N, HID), f32),
        mesh=plsc.VectorSubcoreMesh(core_axis_name="c", subcore_axis_name="s",
                                    num_cores=2, num_subcores=NSUB),
        compiler_params=pltpu.CompilerParams(needs_layout_passes=False,
                                             use_tc_tiling_on_sc=False),
        scratch_types=[
            pltpu.VMEM((RPT, 96), jnp.int32),
            pltpu.VMEM((RPT, 96), f32),
            pltpu.VMEM((96, HID), f32),
            pltpu.VMEM((96, HID), f32),
            pltpu.VMEM((RPT, HI), jnp.int32),
            pltpu.SemaphoreType.DMA,
            pltpu.SemaphoreType.DMA,
            pltpu.SemaphoreType.DMA,
        ],
    )
    outT = sc_out(h.reshape(BB * K, HID), sim3, seg4)
    return outT.reshape(BB, N, HID).transpose(0, 2, 1).reshape(BB, HID, HH, WW)
